# baseline (speedup 1.0000x reference)
import functools

import jax
import jax.numpy as jnp
from jax import lax
from jax.experimental import pallas as pl
from jax.experimental.pallas import tpu as pltpu
from jax.experimental.pallas import tpu_sc as plsc

EMBED_DIM = 384
NUM_HEADS = 8
QHD = 24
QDIM = NUM_HEADS * QHD
BS = 8
TOPK = 32
NBS = BS * BS
NKK = NBS + TOPK
KPAD = 256

SC_CORES = 2
SC_SUBCORES = 16
SC_WORKERS = SC_CORES * SC_SUBCORES


def _proj_body(x_ref, wq_ref, wk_ref, bq_ref, bk_ref, q_ref, k_ref):
    x = x_ref[...]
    q_ref[...] = (
        jnp.dot(x, wq_ref[...], preferred_element_type=jnp.float32) + bq_ref[...]
    )
    k_ref[...] = (
        jnp.dot(x, wk_ref[...], preferred_element_type=jnp.float32) + bk_ref[...]
    )


def _project(x2d, wq, wk, bq, bk, rows_per_step=1024):
    n_rows = x2d.shape[0]
    grid = (n_rows // rows_per_step,)
    q, k = pl.pallas_call(
        _proj_body,
        grid=grid,
        in_specs=[
            pl.BlockSpec((rows_per_step, EMBED_DIM), lambda i: (i, 0)),
            pl.BlockSpec((EMBED_DIM, QDIM), lambda i: (0, 0)),
            pl.BlockSpec((EMBED_DIM, KPAD), lambda i: (0, 0)),
            pl.BlockSpec((1, QDIM), lambda i: (0, 0)),
            pl.BlockSpec((1, KPAD), lambda i: (0, 0)),
        ],
        out_specs=[
            pl.BlockSpec((rows_per_step, QDIM), lambda i: (i, 0)),
            pl.BlockSpec((rows_per_step, KPAD), lambda i: (i, 0)),
        ],
        out_shape=[
            jax.ShapeDtypeStruct((n_rows, QDIM), jnp.float32),
            jax.ShapeDtypeStruct((n_rows, KPAD), jnp.float32),
        ],
    )(x2d, wq, wk, bq, bk)
    return q, k


def _make_sc_gather(n_idx, chunk):
    per_w = n_idx // SC_WORKERS
    n_chunks = per_w // chunk
    mesh = plsc.VectorSubcoreMesh(core_axis_name="c", subcore_axis_name="s")

    @functools.partial(
        pl.kernel,
        mesh=mesh,
        out_type=jax.ShapeDtypeStruct((n_idx, KPAD), jnp.float32),
        scratch_types=[
            pltpu.VMEM((per_w,), jnp.int32),
            pltpu.VMEM((chunk, KPAD), jnp.float32),
            pltpu.VMEM((chunk, KPAD), jnp.float32),
            pltpu.SemaphoreType.DMA,
            pltpu.SemaphoreType.DMA,
        ],
    )
    def gather(table_hbm, idx_hbm, out_hbm, idx_v, rows_a, rows_b, sem_a, sem_b):
        wid = lax.axis_index("s") * SC_CORES + lax.axis_index("c")
        base = wid * per_w
        pltpu.sync_copy(idx_hbm.at[pl.ds(base, per_w)], idx_v)
        bufs = (rows_a, rows_b)
        sems = (sem_a, sem_b)
        cps = []
        for c in range(min(2, n_chunks)):
            cp = pltpu.make_async_copy(
                table_hbm.at[idx_v.at[pl.ds(c * chunk, chunk)]],
                bufs[c % 2],
                sems[c % 2],
            )
            cp.start()
            cps.append(cp)
        for c in range(n_chunks):
            cps[c].wait()
            pltpu.sync_copy(bufs[c % 2], out_hbm.at[pl.ds(base + c * chunk, chunk)])
            nxt = c + 2
            if nxt < n_chunks:
                cp = pltpu.make_async_copy(
                    table_hbm.at[idx_v.at[pl.ds(nxt * chunk, chunk)]],
                    bufs[nxt % 2],
                    sems[nxt % 2],
                )
                cp.start()
                cps.append(cp)

    return gather


def _attn_body(q_ref, k_ref, sel_ref, w_ref, out_ref, *, nbw):
    q_slab = q_ref[0, 0]
    k_slab = k_ref[0, 0]
    for j in range(nbw):
        qj = q_slab[:, j * BS:(j + 1) * BS, :].reshape(NBS, QDIM)
        kj = k_slab[:, j * BS:(j + 1) * BS, :QDIM].reshape(NBS, QDIM)
        selj = sel_ref[0, 0, j, :, :QDIM]
        wj = w_ref[0, 0, j]
        kk = jnp.concatenate([kj, selj * wj], axis=0)
        heads = []
        for h in range(NUM_HEADS):
            qh = qj[:, h * QHD:(h + 1) * QHD]
            kh = kk[:, h * QHD:(h + 1) * QHD]
            s = lax.dot_general(
                qh, kh, (((1,), (1,)), ((), ())),
                preferred_element_type=jnp.float32,
            )
            heads.append(s)
        e = jnp.exp(jnp.stack(heads, axis=0))
        e2 = e.reshape(NUM_HEADS * NBS, NKK)
        denom = lax.dot_general(
            e2, jnp.ones((NKK, NKK), jnp.float32),
            (((1,), (0,)), ((), ())),
            preferred_element_type=jnp.float32,
        )
        out_ref[:, 0, j] = (e2 / denom).reshape(NUM_HEADS, NBS, NKK)


def _attention(q5, k5, sel5, w5, B, nbh, nbw):
    grid = (B, nbh)
    out = pl.pallas_call(
        functools.partial(_attn_body, nbw=nbw),
        grid=grid,
        in_specs=[
            pl.BlockSpec((1, 1, BS, nbw * BS, QDIM), lambda b, r: (b, r, 0, 0, 0)),
            pl.BlockSpec((1, 1, BS, nbw * BS, KPAD), lambda b, r: (b, r, 0, 0, 0)),
            pl.BlockSpec((1, 1, nbw, TOPK, KPAD), lambda b, r: (b, r, 0, 0, 0)),
            pl.BlockSpec((1, 1, nbw, TOPK, 1), lambda b, r: (b, r, 0, 0, 0)),
        ],
        out_specs=pl.BlockSpec(
            (NUM_HEADS, 1, nbw, NBS, NKK),
            lambda b, r: (0, b, r, 0, 0),
        ),
        out_shape=jax.ShapeDtypeStruct(
            (NUM_HEADS, B, nbh * nbw, NBS, NKK), jnp.float32
        ),
    )(q5, k5, sel5, w5)
    return out


TBLK = 128


def _xpose_body(a_ref, b_ref):
    a = a_ref[0, 0]
    b_ref[0, 0] = jnp.transpose(a, (1, 2, 0))


def _to_blk_minor(a, B, nbt):
    nt = -(-nbt // TBLK)
    return pl.pallas_call(
        _xpose_body,
        grid=(B, NUM_HEADS, nt),
        in_specs=[
            pl.BlockSpec((1, 1, TBLK, NBS, NKK), lambda b, h, t: (h, b, t, 0, 0)),
        ],
        out_specs=pl.BlockSpec(
            (1, 1, NBS, NKK, TBLK), lambda b, h, t: (h, b, 0, 0, t)
        ),
        out_shape=jax.ShapeDtypeStruct((NUM_HEADS, B, NBS, NKK, nbt), jnp.float32),
    )(a)


def kernel(x, indexes, weights, W_in, b_in):
    B, H, W, _ = x.shape
    nbh, nbw = H // BS, W // BS
    nbt = nbh * nbw
    num_tokens = H * W

    wq = W_in[:QDIM].T
    wk = jnp.pad(W_in[QDIM:].T, ((0, 0), (0, KPAD - QDIM)))
    bq = b_in[:QDIM].reshape(1, QDIM)
    bk = jnp.pad(b_in[QDIM:], (0, KPAD - QDIM)).reshape(1, KPAD)

    x2d = x.reshape(B * num_tokens, EMBED_DIM)
    q, k = _project(x2d, wq, wk, bq, bk)

    idx = (indexes + (jnp.arange(B, dtype=jnp.int32) * num_tokens)[:, None, None])
    idx = idx.reshape(-1)
    sel = _make_sc_gather(idx.shape[0], chunk=224)(k, idx)

    q5 = q.reshape(B, nbh, BS, W, QDIM)
    k5 = k.reshape(B, nbh, BS, W, KPAD)
    sel5 = sel.reshape(B, nbh, nbw, TOPK, KPAD)
    w5 = weights.reshape(B, nbh, nbw, TOPK, 1)

    att = _attention(q5, k5, sel5, w5, B, nbh, nbw)
    return jnp.transpose(_to_blk_minor(att, B, nbt), (0, 1, 4, 2, 3))


# speedup vs baseline: 1.0159x; 1.0159x over previous
import functools

import jax
import jax.numpy as jnp
from jax import lax
from jax.experimental import pallas as pl
from jax.experimental.pallas import tpu as pltpu
from jax.experimental.pallas import tpu_sc as plsc

EMBED_DIM = 384
NUM_HEADS = 8
QHD = 24
QDIM = NUM_HEADS * QHD
BS = 8
TOPK = 32
NBS = BS * BS
NKK = NBS + TOPK
NKPAD = 128
KPAD = 256

SC_CORES = 2
SC_SUBCORES = 16
SC_WORKERS = SC_CORES * SC_SUBCORES


def _proj_body(x_ref, wq_ref, wk_ref, bq_ref, bk_ref, q_ref, k_ref):
    x = x_ref[...]
    q_ref[...] = (
        jnp.dot(x, wq_ref[...], preferred_element_type=jnp.float32) + bq_ref[...]
    )
    k_ref[...] = (
        jnp.dot(x, wk_ref[...], preferred_element_type=jnp.float32) + bk_ref[...]
    )


def _project(x2d, wq, wk, bq, bk, rows_per_step=1024):
    n_rows = x2d.shape[0]
    grid = (n_rows // rows_per_step,)
    q, k = pl.pallas_call(
        _proj_body,
        grid=grid,
        in_specs=[
            pl.BlockSpec((rows_per_step, EMBED_DIM), lambda i: (i, 0)),
            pl.BlockSpec((EMBED_DIM, QDIM), lambda i: (0, 0)),
            pl.BlockSpec((EMBED_DIM, KPAD), lambda i: (0, 0)),
            pl.BlockSpec((1, QDIM), lambda i: (0, 0)),
            pl.BlockSpec((1, KPAD), lambda i: (0, 0)),
        ],
        out_specs=[
            pl.BlockSpec((rows_per_step, QDIM), lambda i: (i, 0)),
            pl.BlockSpec((rows_per_step, KPAD), lambda i: (i, 0)),
        ],
        out_shape=[
            jax.ShapeDtypeStruct((n_rows, QDIM), jnp.float32),
            jax.ShapeDtypeStruct((n_rows, KPAD), jnp.float32),
        ],
    )(x2d, wq, wk, bq, bk)
    return q, k


def _make_sc_gather(n_idx, chunk):
    per_w = n_idx // SC_WORKERS
    n_chunks = per_w // chunk
    mesh = plsc.VectorSubcoreMesh(core_axis_name="c", subcore_axis_name="s")

    @functools.partial(
        pl.kernel,
        mesh=mesh,
        out_type=jax.ShapeDtypeStruct((n_idx, KPAD), jnp.float32),
        scratch_types=[
            pltpu.VMEM((per_w,), jnp.int32),
            pltpu.VMEM((chunk, KPAD), jnp.float32),
            pltpu.VMEM((chunk, KPAD), jnp.float32),
            pltpu.SemaphoreType.DMA,
            pltpu.SemaphoreType.DMA,
        ],
    )
    def gather(table_hbm, idx_hbm, out_hbm, idx_v, rows_a, rows_b, sem_a, sem_b):
        wid = lax.axis_index("s") * SC_CORES + lax.axis_index("c")
        base = wid * per_w
        pltpu.sync_copy(idx_hbm.at[pl.ds(base, per_w)], idx_v)
        bufs = (rows_a, rows_b)
        sems = (sem_a, sem_b)
        cps = []
        for c in range(min(2, n_chunks)):
            cp = pltpu.make_async_copy(
                table_hbm.at[idx_v.at[pl.ds(c * chunk, chunk)]],
                bufs[c % 2],
                sems[c % 2],
            )
            cp.start()
            cps.append(cp)
        for c in range(n_chunks):
            cps[c].wait()
            pltpu.sync_copy(bufs[c % 2], out_hbm.at[pl.ds(base + c * chunk, chunk)])
            nxt = c + 2
            if nxt < n_chunks:
                cp = pltpu.make_async_copy(
                    table_hbm.at[idx_v.at[pl.ds(nxt * chunk, chunk)]],
                    bufs[nxt % 2],
                    sems[nxt % 2],
                )
                cp.start()
                cps.append(cp)

    return gather


def _attn_body(q_ref, k_ref, sel_ref, w_ref, out_ref, *, nbw):
    q_slab = q_ref[0, 0]
    k_slab = k_ref[0, 0]
    for j in range(nbw):
        qj = q_slab[:, j * BS:(j + 1) * BS, :].reshape(NBS, QDIM)
        kj = k_slab[:, j * BS:(j + 1) * BS, :QDIM].reshape(NBS, QDIM)
        selj = sel_ref[0, 0, j, :, :QDIM]
        wj = w_ref[0, 0, j]
        kk = jnp.concatenate(
            [kj, selj * wj, jnp.zeros((NKPAD - NKK, QDIM), jnp.float32)],
            axis=0,
        )
        heads = []
        for h in range(NUM_HEADS):
            qh = qj[:, h * QHD:(h + 1) * QHD]
            kh = kk[:, h * QHD:(h + 1) * QHD]
            s = lax.dot_general(
                qh, kh, (((1,), (1,)), ((), ())),
                preferred_element_type=jnp.float32,
            )
            heads.append(s)
        e = jnp.exp(jnp.stack(heads, axis=0))
        e2 = e.reshape(NUM_HEADS * NBS, NKPAD)
        ones_mask = jnp.concatenate(
            [jnp.ones((NKK, NKPAD), jnp.float32),
             jnp.zeros((NKPAD - NKK, NKPAD), jnp.float32)],
            axis=0,
        )
        denom = lax.dot_general(
            e2, ones_mask,
            (((1,), (0,)), ((), ())),
            preferred_element_type=jnp.float32,
        )
        out_ref[:, 0, j] = (e2 / denom).reshape(NUM_HEADS, NBS, NKPAD)


def _attention(q5, k5, sel5, w5, B, nbh, nbw):
    grid = (B, nbh)
    out = pl.pallas_call(
        functools.partial(_attn_body, nbw=nbw),
        grid=grid,
        in_specs=[
            pl.BlockSpec((1, 1, BS, nbw * BS, QDIM), lambda b, r: (b, r, 0, 0, 0)),
            pl.BlockSpec((1, 1, BS, nbw * BS, KPAD), lambda b, r: (b, r, 0, 0, 0)),
            pl.BlockSpec((1, 1, nbw, TOPK, KPAD), lambda b, r: (b, r, 0, 0, 0)),
            pl.BlockSpec((1, 1, nbw, TOPK, 1), lambda b, r: (b, r, 0, 0, 0)),
        ],
        out_specs=pl.BlockSpec(
            (NUM_HEADS, 1, nbw, NBS, NKPAD),
            lambda b, r: (0, b, r, 0, 0),
        ),
        out_shape=jax.ShapeDtypeStruct(
            (NUM_HEADS, B, nbh * nbw, NBS, NKPAD), jnp.float32
        ),
    )(q5, k5, sel5, w5)
    return out


TBLK = 128


def _xpose_body(a_ref, b_ref):
    a = a_ref[0, 0]
    t = jnp.transpose(a.reshape(TBLK, NBS * NKPAD))
    b_ref[0, 0] = t.reshape(NBS, NKPAD, TBLK)[:, :NKK, :]


def _to_blk_minor(a, B, nbt):
    nt = -(-nbt // TBLK)
    return pl.pallas_call(
        _xpose_body,
        grid=(B, NUM_HEADS, nt),
        in_specs=[
            pl.BlockSpec((1, 1, TBLK, NBS, NKPAD), lambda b, h, t: (h, b, t, 0, 0)),
        ],
        out_specs=pl.BlockSpec(
            (1, 1, NBS, NKK, TBLK), lambda b, h, t: (h, b, 0, 0, t)
        ),
        out_shape=jax.ShapeDtypeStruct((NUM_HEADS, B, NBS, NKK, nbt), jnp.float32),
    )(a)


def kernel(x, indexes, weights, W_in, b_in):
    B, H, W, _ = x.shape
    nbh, nbw = H // BS, W // BS
    nbt = nbh * nbw
    num_tokens = H * W

    wq = W_in[:QDIM].T
    wk = jnp.pad(W_in[QDIM:].T, ((0, 0), (0, KPAD - QDIM)))
    bq = b_in[:QDIM].reshape(1, QDIM)
    bk = jnp.pad(b_in[QDIM:], (0, KPAD - QDIM)).reshape(1, KPAD)

    x2d = x.reshape(B * num_tokens, EMBED_DIM)
    q, k = _project(x2d, wq, wk, bq, bk)

    idx = (indexes + (jnp.arange(B, dtype=jnp.int32) * num_tokens)[:, None, None])
    idx = idx.reshape(-1)
    sel = _make_sc_gather(idx.shape[0], chunk=224)(k, idx)

    q5 = q.reshape(B, nbh, BS, W, QDIM)
    k5 = k.reshape(B, nbh, BS, W, KPAD)
    sel5 = sel.reshape(B, nbh, nbw, TOPK, KPAD)
    w5 = weights.reshape(B, nbh, nbw, TOPK, 1)

    att = _attention(q5, k5, sel5, w5, B, nbh, nbw)
    return jnp.transpose(_to_blk_minor(att, B, nbt), (0, 1, 4, 2, 3))


# speedup vs baseline: 1.1577x; 1.1396x over previous
import functools

import jax
import jax.numpy as jnp
from jax import lax
from jax.experimental import pallas as pl
from jax.experimental.pallas import tpu as pltpu
from jax.experimental.pallas import tpu_sc as plsc

EMBED_DIM = 384
NUM_HEADS = 8
QHD = 24
QDIM = NUM_HEADS * QHD
BS = 8
TOPK = 32
NBS = BS * BS
NKK = NBS + TOPK
NKPAD = 128
KPAD = 256

SC_CORES = 2
SC_SUBCORES = 16
SC_WORKERS = SC_CORES * SC_SUBCORES


def _proj_body(x_ref, wq_ref, wk_ref, bq_ref, bk_ref, q_ref, k_ref):
    x = x_ref[...]
    q_ref[...] = (
        jnp.dot(x, wq_ref[...], preferred_element_type=jnp.float32) + bq_ref[...]
    )
    k_ref[...] = (
        jnp.dot(x, wk_ref[...], preferred_element_type=jnp.float32) + bk_ref[...]
    )


def _project(x2d, wq, wk, bq, bk, rows_per_step=1024):
    n_rows = x2d.shape[0]
    grid = (n_rows // rows_per_step,)
    q, k = pl.pallas_call(
        _proj_body,
        grid=grid,
        in_specs=[
            pl.BlockSpec((rows_per_step, EMBED_DIM), lambda i: (i, 0)),
            pl.BlockSpec((EMBED_DIM, QDIM), lambda i: (0, 0)),
            pl.BlockSpec((EMBED_DIM, KPAD), lambda i: (0, 0)),
            pl.BlockSpec((1, QDIM), lambda i: (0, 0)),
            pl.BlockSpec((1, KPAD), lambda i: (0, 0)),
        ],
        out_specs=[
            pl.BlockSpec((rows_per_step, QDIM), lambda i: (i, 0)),
            pl.BlockSpec((rows_per_step, KPAD), lambda i: (i, 0)),
        ],
        out_shape=[
            jax.ShapeDtypeStruct((n_rows, QDIM), jnp.float32),
            jax.ShapeDtypeStruct((n_rows, KPAD), jnp.float32),
        ],
    )(x2d, wq, wk, bq, bk)
    return q, k


def _make_sc_gather(n_idx, chunk):
    per_w = n_idx // SC_WORKERS
    n_chunks = per_w // chunk
    mesh = plsc.VectorSubcoreMesh(core_axis_name="c", subcore_axis_name="s")

    @functools.partial(
        pl.kernel,
        mesh=mesh,
        out_type=jax.ShapeDtypeStruct((n_idx, KPAD), jnp.float32),
        scratch_types=[
            pltpu.VMEM((per_w,), jnp.int32),
            pltpu.VMEM((chunk, KPAD), jnp.float32),
            pltpu.VMEM((chunk, KPAD), jnp.float32),
            pltpu.SemaphoreType.DMA,
            pltpu.SemaphoreType.DMA,
        ],
    )
    def gather(table_hbm, idx_hbm, out_hbm, idx_v, rows_a, rows_b, sem_a, sem_b):
        wid = lax.axis_index("s") * SC_CORES + lax.axis_index("c")
        base = wid * per_w
        pltpu.sync_copy(idx_hbm.at[pl.ds(base, per_w)], idx_v)
        bufs = (rows_a, rows_b)
        sems = (sem_a, sem_b)
        cps = []
        for c in range(min(2, n_chunks)):
            cp = pltpu.make_async_copy(
                table_hbm.at[idx_v.at[pl.ds(c * chunk, chunk)]],
                bufs[c % 2],
                sems[c % 2],
            )
            cp.start()
            cps.append(cp)
        for c in range(n_chunks):
            cps[c].wait()
            pltpu.sync_copy(bufs[c % 2], out_hbm.at[pl.ds(base + c * chunk, chunk)])
            nxt = c + 2
            if nxt < n_chunks:
                cp = pltpu.make_async_copy(
                    table_hbm.at[idx_v.at[pl.ds(nxt * chunk, chunk)]],
                    bufs[nxt % 2],
                    sems[nxt % 2],
                )
                cp.start()
                cps.append(cp)

    return gather


def _attn_body(q_ref, k_ref, sel_ref, w_ref, out_ref, *, nbw):
    q_slab = q_ref[0, 0]
    k_slab = k_ref[0, 0]
    for j in range(nbw):
        qj = q_slab[:, j * BS:(j + 1) * BS, :].reshape(NBS, QDIM)
        kj = k_slab[:, j * BS:(j + 1) * BS, :QDIM].reshape(NBS, QDIM)
        selj = sel_ref[0, 0, j, :, :QDIM]
        wj = w_ref[0, 0, j]
        kk = jnp.concatenate([kj, selj * wj], axis=0)
        heads = []
        for h in range(NUM_HEADS):
            qh = qj[:, h * QHD:(h + 1) * QHD]
            kh = kk[:, h * QHD:(h + 1) * QHD]
            s = lax.dot_general(
                qh, kh, (((1,), (1,)), ((), ())),
                preferred_element_type=jnp.float32,
            )
            heads.append(s)
        e = jnp.exp(jnp.stack(heads, axis=0))
        e2 = e.reshape(NUM_HEADS * NBS, NKK)
        denom = lax.dot_general(
            e2, jnp.ones((NKK, NKK), jnp.float32),
            (((1,), (0,)), ((), ())),
            preferred_element_type=jnp.float32,
        )
        out_ref[:, 0, j] = (e2 / denom).reshape(NUM_HEADS, NBS, NKK)


def _attention(q5, k5, sel5, w5, B, nbh, nbw):
    grid = (B, nbh)
    out = pl.pallas_call(
        functools.partial(_attn_body, nbw=nbw),
        grid=grid,
        in_specs=[
            pl.BlockSpec((1, 1, BS, nbw * BS, QDIM), lambda b, r: (b, r, 0, 0, 0)),
            pl.BlockSpec((1, 1, BS, nbw * BS, KPAD), lambda b, r: (b, r, 0, 0, 0)),
            pl.BlockSpec((1, 1, nbw, TOPK, KPAD), lambda b, r: (b, r, 0, 0, 0)),
            pl.BlockSpec((1, 1, nbw, TOPK, 1), lambda b, r: (b, r, 0, 0, 0)),
        ],
        out_specs=pl.BlockSpec(
            (NUM_HEADS, 1, nbw, NBS, NKK),
            lambda b, r: (0, b, r, 0, 0),
        ),
        out_shape=jax.ShapeDtypeStruct(
            (NUM_HEADS, B, nbh * nbw, NBS, NKK), jnp.float32
        ),
    )(q5, k5, sel5, w5)
    return out


TBLK = 128


def _xpose_body(a_ref, b_ref):
    a = a_ref[0, 0]
    ap = jnp.concatenate(
        [a, jnp.zeros((TBLK, NBS, NKPAD - NKK), jnp.float32)], axis=-1
    )
    t = jnp.transpose(ap.reshape(TBLK, NBS * NKPAD))
    b_ref[0, 0] = t.reshape(NBS, NKPAD, TBLK)[:, :NKK, :]


def _to_blk_minor(a, B, nbt):
    nt = -(-nbt // TBLK)
    return pl.pallas_call(
        _xpose_body,
        grid=(B, NUM_HEADS, nt),
        in_specs=[
            pl.BlockSpec((1, 1, TBLK, NBS, NKK), lambda b, h, t: (h, b, t, 0, 0)),
        ],
        out_specs=pl.BlockSpec(
            (1, 1, NBS, NKK, TBLK), lambda b, h, t: (h, b, 0, 0, t)
        ),
        out_shape=jax.ShapeDtypeStruct((NUM_HEADS, B, NBS, NKK, nbt), jnp.float32),
    )(a)


def kernel(x, indexes, weights, W_in, b_in):
    B, H, W, _ = x.shape
    nbh, nbw = H // BS, W // BS
    nbt = nbh * nbw
    num_tokens = H * W

    wq = W_in[:QDIM].T
    wk = jnp.pad(W_in[QDIM:].T, ((0, 0), (0, KPAD - QDIM)))
    bq = b_in[:QDIM].reshape(1, QDIM)
    bk = jnp.pad(b_in[QDIM:], (0, KPAD - QDIM)).reshape(1, KPAD)

    x2d = x.reshape(B * num_tokens, EMBED_DIM)
    q, k = _project(x2d, wq, wk, bq, bk)

    idx = (indexes + (jnp.arange(B, dtype=jnp.int32) * num_tokens)[:, None, None])
    idx = idx.reshape(-1)
    sel = _make_sc_gather(idx.shape[0], chunk=224)(k, idx)

    q5 = q.reshape(B, nbh, BS, W, QDIM)
    k5 = k.reshape(B, nbh, BS, W, KPAD)
    sel5 = sel.reshape(B, nbh, nbw, TOPK, KPAD)
    w5 = weights.reshape(B, nbh, nbw, TOPK, 1)

    att = _attention(q5, k5, sel5, w5, B, nbh, nbw)
    return jnp.transpose(_to_blk_minor(att, B, nbt), (0, 1, 4, 2, 3))


# speedup vs baseline: 1.2027x; 1.0389x over previous
import functools

import jax
import jax.numpy as jnp
from jax import lax
from jax.experimental import pallas as pl
from jax.experimental.pallas import tpu as pltpu
from jax.experimental.pallas import tpu_sc as plsc

EMBED_DIM = 384
NUM_HEADS = 8
QHD = 24
QDIM = NUM_HEADS * QHD
BS = 8
TOPK = 32
NBS = BS * BS
NKK = NBS + TOPK
NKPAD = 128
KPAD = 256

SC_CORES = 2
SC_SUBCORES = 16
SC_WORKERS = SC_CORES * SC_SUBCORES


def _proj_body(x_ref, wq_ref, wk_ref, bq_ref, bk_ref, q_ref, k_ref):
    x = x_ref[...]
    q_ref[...] = (
        jnp.dot(x, wq_ref[...], preferred_element_type=jnp.float32) + bq_ref[...]
    )
    k_ref[...] = (
        jnp.dot(x, wk_ref[...], preferred_element_type=jnp.float32) + bk_ref[...]
    )


def _project(x2d, wq, wk, bq, bk, rows_per_step=2048):
    n_rows = x2d.shape[0]
    grid = (n_rows // rows_per_step,)
    q, k = pl.pallas_call(
        _proj_body,
        grid=grid,
        in_specs=[
            pl.BlockSpec((rows_per_step, EMBED_DIM), lambda i: (i, 0)),
            pl.BlockSpec((EMBED_DIM, QDIM), lambda i: (0, 0)),
            pl.BlockSpec((EMBED_DIM, KPAD), lambda i: (0, 0)),
            pl.BlockSpec((1, QDIM), lambda i: (0, 0)),
            pl.BlockSpec((1, KPAD), lambda i: (0, 0)),
        ],
        out_specs=[
            pl.BlockSpec((rows_per_step, QDIM), lambda i: (i, 0)),
            pl.BlockSpec((rows_per_step, KPAD), lambda i: (i, 0)),
        ],
        out_shape=[
            jax.ShapeDtypeStruct((n_rows, QDIM), jnp.float32),
            jax.ShapeDtypeStruct((n_rows, KPAD), jnp.float32),
        ],
    )(x2d, wq, wk, bq, bk)
    return q, k


def _make_sc_gather(n_idx, chunk):
    per_w = n_idx // SC_WORKERS
    n_chunks = per_w // chunk
    mesh = plsc.VectorSubcoreMesh(core_axis_name="c", subcore_axis_name="s")

    @functools.partial(
        pl.kernel,
        mesh=mesh,
        out_type=jax.ShapeDtypeStruct((n_idx, KPAD), jnp.float32),
        scratch_types=[
            pltpu.VMEM((per_w,), jnp.int32),
            pltpu.VMEM((chunk, KPAD), jnp.float32),
            pltpu.VMEM((chunk, KPAD), jnp.float32),
            pltpu.SemaphoreType.DMA,
            pltpu.SemaphoreType.DMA,
        ],
    )
    def gather(table_hbm, idx_hbm, out_hbm, idx_v, rows_a, rows_b, sem_a, sem_b):
        wid = lax.axis_index("s") * SC_CORES + lax.axis_index("c")
        base = wid * per_w
        pltpu.sync_copy(idx_hbm.at[pl.ds(base, per_w)], idx_v)
        bufs = (rows_a, rows_b)
        sems = (sem_a, sem_b)
        cps = []
        for c in range(min(2, n_chunks)):
            cp = pltpu.make_async_copy(
                table_hbm.at[idx_v.at[pl.ds(c * chunk, chunk)]],
                bufs[c % 2],
                sems[c % 2],
            )
            cp.start()
            cps.append(cp)
        for c in range(n_chunks):
            cps[c].wait()
            pltpu.sync_copy(bufs[c % 2], out_hbm.at[pl.ds(base + c * chunk, chunk)])
            nxt = c + 2
            if nxt < n_chunks:
                cp = pltpu.make_async_copy(
                    table_hbm.at[idx_v.at[pl.ds(nxt * chunk, chunk)]],
                    bufs[nxt % 2],
                    sems[nxt % 2],
                )
                cp.start()
                cps.append(cp)

    return gather


def _attn_body(q_ref, k_ref, sel_ref, w_ref, out_ref, *, nbw):
    q_slab = q_ref[0, 0]
    k_slab = k_ref[0, 0]
    for j in range(nbw):
        qj = q_slab[:, j * BS:(j + 1) * BS, :].reshape(NBS, QDIM)
        kj = k_slab[:, j * BS:(j + 1) * BS, :QDIM].reshape(NBS, QDIM)
        selj = sel_ref[0, 0, j, :, :QDIM]
        wj = w_ref[0, 0, j]
        kk = jnp.concatenate([kj, selj * wj], axis=0)
        heads = []
        for h in range(NUM_HEADS):
            qh = qj[:, h * QHD:(h + 1) * QHD]
            kh = kk[:, h * QHD:(h + 1) * QHD]
            s = lax.dot_general(
                qh, kh, (((1,), (1,)), ((), ())),
                preferred_element_type=jnp.float32,
            )
            heads.append(s)
        e = jnp.exp(jnp.stack(heads, axis=0))
        e2 = e.reshape(NUM_HEADS * NBS, NKK)
        denom = lax.dot_general(
            e2, jnp.ones((NKK, NKK), jnp.float32),
            (((1,), (0,)), ((), ())),
            preferred_element_type=jnp.float32,
        )
        out_ref[:, 0, j] = (e2 / denom).reshape(NUM_HEADS, NBS, NKK)


def _attention(q5, k5, sel5, w5, B, nbh, nbw):
    grid = (B, nbh)
    out = pl.pallas_call(
        functools.partial(_attn_body, nbw=nbw),
        grid=grid,
        in_specs=[
            pl.BlockSpec((1, 1, BS, nbw * BS, QDIM), lambda b, r: (b, r, 0, 0, 0)),
            pl.BlockSpec((1, 1, BS, nbw * BS, KPAD), lambda b, r: (b, r, 0, 0, 0)),
            pl.BlockSpec((1, 1, nbw, TOPK, KPAD), lambda b, r: (b, r, 0, 0, 0)),
            pl.BlockSpec((1, 1, nbw, TOPK, 1), lambda b, r: (b, r, 0, 0, 0)),
        ],
        out_specs=pl.BlockSpec(
            (NUM_HEADS, 1, nbw, NBS, NKK),
            lambda b, r: (0, b, r, 0, 0),
        ),
        out_shape=jax.ShapeDtypeStruct(
            (NUM_HEADS, B, nbh * nbw, NBS, NKK), jnp.float32
        ),
    )(q5, k5, sel5, w5)
    return out


TBLK = 128


def _xpose_body(a_ref, b_ref):
    a = a_ref[0, 0]
    ap = jnp.concatenate(
        [a, jnp.zeros((TBLK, NBS, NKPAD - NKK), jnp.float32)], axis=-1
    )
    t = jnp.transpose(ap.reshape(TBLK, NBS * NKPAD))
    b_ref[0, 0] = t.reshape(NBS, NKPAD, TBLK)[:, :NKK, :]


def _to_blk_minor(a, B, nbt):
    nt = -(-nbt // TBLK)
    return pl.pallas_call(
        _xpose_body,
        grid=(B, NUM_HEADS, nt),
        in_specs=[
            pl.BlockSpec((1, 1, TBLK, NBS, NKK), lambda b, h, t: (h, b, t, 0, 0)),
        ],
        out_specs=pl.BlockSpec(
            (1, 1, NBS, NKK, TBLK), lambda b, h, t: (h, b, 0, 0, t)
        ),
        out_shape=jax.ShapeDtypeStruct((NUM_HEADS, B, NBS, NKK, nbt), jnp.float32),
    )(a)


def kernel(x, indexes, weights, W_in, b_in):
    B, H, W, _ = x.shape
    nbh, nbw = H // BS, W // BS
    nbt = nbh * nbw
    num_tokens = H * W

    wq = W_in[:QDIM].T
    wk = jnp.pad(W_in[QDIM:].T, ((0, 0), (0, KPAD - QDIM)))
    bq = b_in[:QDIM].reshape(1, QDIM)
    bk = jnp.pad(b_in[QDIM:], (0, KPAD - QDIM)).reshape(1, KPAD)

    x2d = x.reshape(B * num_tokens, EMBED_DIM)
    q, k = _project(x2d, wq, wk, bq, bk)

    idx = (indexes + (jnp.arange(B, dtype=jnp.int32) * num_tokens)[:, None, None])
    idx = idx.reshape(-1)
    sel = _make_sc_gather(idx.shape[0], chunk=224)(k, idx)

    q5 = q.reshape(B, nbh, BS, W, QDIM)
    k5 = k.reshape(B, nbh, BS, W, KPAD)
    sel5 = sel.reshape(B, nbh, nbw, TOPK, KPAD)
    w5 = weights.reshape(B, nbh, nbw, TOPK, 1)

    att = _attention(q5, k5, sel5, w5, B, nbh, nbw)
    return jnp.transpose(_to_blk_minor(att, B, nbt), (0, 1, 4, 2, 3))


# speedup vs baseline: 1.2173x; 1.0121x over previous
import functools

import jax
import jax.numpy as jnp
from jax import lax
from jax.experimental import pallas as pl
from jax.experimental.pallas import tpu as pltpu
from jax.experimental.pallas import tpu_sc as plsc

EMBED_DIM = 384
NUM_HEADS = 8
QHD = 24
QDIM = NUM_HEADS * QHD
BS = 8
TOPK = 32
NBS = BS * BS
NKK = NBS + TOPK
NKPAD = 128
KPAD = 256

SC_CORES = 2
SC_SUBCORES = 16
SC_WORKERS = SC_CORES * SC_SUBCORES


def _proj_body(x_ref, wq_ref, wk_ref, bq_ref, bk_ref, q_ref, k_ref):
    x = x_ref[...]
    q_ref[...] = (
        jnp.dot(x, wq_ref[...], preferred_element_type=jnp.float32) + bq_ref[...]
    )
    k_ref[...] = (
        jnp.dot(x, wk_ref[...], preferred_element_type=jnp.float32) + bk_ref[...]
    )


def _project(x2d, wq, wk, bq, bk, rows_per_step=4096):
    n_rows = x2d.shape[0]
    grid = (n_rows // rows_per_step,)
    q, k = pl.pallas_call(
        _proj_body,
        grid=grid,
        in_specs=[
            pl.BlockSpec((rows_per_step, EMBED_DIM), lambda i: (i, 0)),
            pl.BlockSpec((EMBED_DIM, QDIM), lambda i: (0, 0)),
            pl.BlockSpec((EMBED_DIM, KPAD), lambda i: (0, 0)),
            pl.BlockSpec((1, QDIM), lambda i: (0, 0)),
            pl.BlockSpec((1, KPAD), lambda i: (0, 0)),
        ],
        out_specs=[
            pl.BlockSpec((rows_per_step, QDIM), lambda i: (i, 0)),
            pl.BlockSpec((rows_per_step, KPAD), lambda i: (i, 0)),
        ],
        out_shape=[
            jax.ShapeDtypeStruct((n_rows, QDIM), jnp.float32),
            jax.ShapeDtypeStruct((n_rows, KPAD), jnp.float32),
        ],
    )(x2d, wq, wk, bq, bk)
    return q, k


def _make_sc_gather(n_idx, chunk):
    per_w = n_idx // SC_WORKERS
    n_chunks = per_w // chunk
    mesh = plsc.VectorSubcoreMesh(core_axis_name="c", subcore_axis_name="s")

    @functools.partial(
        pl.kernel,
        mesh=mesh,
        out_type=jax.ShapeDtypeStruct((n_idx, KPAD), jnp.float32),
        scratch_types=[
            pltpu.VMEM((per_w,), jnp.int32),
            pltpu.VMEM((chunk, KPAD), jnp.float32),
            pltpu.VMEM((chunk, KPAD), jnp.float32),
            pltpu.SemaphoreType.DMA,
            pltpu.SemaphoreType.DMA,
        ],
    )
    def gather(table_hbm, idx_hbm, out_hbm, idx_v, rows_a, rows_b, sem_a, sem_b):
        wid = lax.axis_index("s") * SC_CORES + lax.axis_index("c")
        base = wid * per_w
        pltpu.sync_copy(idx_hbm.at[pl.ds(base, per_w)], idx_v)
        bufs = (rows_a, rows_b)
        sems = (sem_a, sem_b)
        cps = []
        for c in range(min(2, n_chunks)):
            cp = pltpu.make_async_copy(
                table_hbm.at[idx_v.at[pl.ds(c * chunk, chunk)]],
                bufs[c % 2],
                sems[c % 2],
            )
            cp.start()
            cps.append(cp)
        for c in range(n_chunks):
            cps[c].wait()
            pltpu.sync_copy(bufs[c % 2], out_hbm.at[pl.ds(base + c * chunk, chunk)])
            nxt = c + 2
            if nxt < n_chunks:
                cp = pltpu.make_async_copy(
                    table_hbm.at[idx_v.at[pl.ds(nxt * chunk, chunk)]],
                    bufs[nxt % 2],
                    sems[nxt % 2],
                )
                cp.start()
                cps.append(cp)

    return gather


def _attn_body(q_ref, k_ref, sel_ref, w_ref, out_ref, *, nbw):
    q_slab = q_ref[0, 0]
    k_slab = k_ref[0, 0]
    for j in range(nbw):
        qj = q_slab[:, j * BS:(j + 1) * BS, :].reshape(NBS, QDIM)
        kj = k_slab[:, j * BS:(j + 1) * BS, :QDIM].reshape(NBS, QDIM)
        selj = sel_ref[0, 0, j, :, :QDIM]
        wj = w_ref[0, 0, j]
        kk = jnp.concatenate([kj, selj * wj], axis=0)
        heads = []
        for h in range(NUM_HEADS):
            qh = qj[:, h * QHD:(h + 1) * QHD]
            kh = kk[:, h * QHD:(h + 1) * QHD]
            s = lax.dot_general(
                qh, kh, (((1,), (1,)), ((), ())),
                preferred_element_type=jnp.float32,
            )
            heads.append(s)
        e = jnp.exp(jnp.stack(heads, axis=0))
        e2 = e.reshape(NUM_HEADS * NBS, NKK)
        denom = lax.dot_general(
            e2, jnp.ones((NKK, NKK), jnp.float32),
            (((1,), (0,)), ((), ())),
            preferred_element_type=jnp.float32,
        )
        out_ref[:, 0, j] = (e2 / denom).reshape(NUM_HEADS, NBS, NKK)


def _attention(q5, k5, sel5, w5, B, nbh, nbw):
    grid = (B, nbh)
    out = pl.pallas_call(
        functools.partial(_attn_body, nbw=nbw),
        grid=grid,
        in_specs=[
            pl.BlockSpec((1, 1, BS, nbw * BS, QDIM), lambda b, r: (b, r, 0, 0, 0)),
            pl.BlockSpec((1, 1, BS, nbw * BS, KPAD), lambda b, r: (b, r, 0, 0, 0)),
            pl.BlockSpec((1, 1, nbw, TOPK, KPAD), lambda b, r: (b, r, 0, 0, 0)),
            pl.BlockSpec((1, 1, nbw, TOPK, 1), lambda b, r: (b, r, 0, 0, 0)),
        ],
        out_specs=pl.BlockSpec(
            (NUM_HEADS, 1, nbw, NBS, NKK),
            lambda b, r: (0, b, r, 0, 0),
        ),
        out_shape=jax.ShapeDtypeStruct(
            (NUM_HEADS, B, nbh * nbw, NBS, NKK), jnp.float32
        ),
    )(q5, k5, sel5, w5)
    return out


TBLK = 128


def _xpose_body(a_ref, b_ref):
    a = a_ref[0, 0]
    ap = jnp.concatenate(
        [a, jnp.zeros((TBLK, NBS, NKPAD - NKK), jnp.float32)], axis=-1
    )
    t = jnp.transpose(ap.reshape(TBLK, NBS * NKPAD))
    b_ref[0, 0] = t.reshape(NBS, NKPAD, TBLK)[:, :NKK, :]


def _to_blk_minor(a, B, nbt):
    nt = -(-nbt // TBLK)
    return pl.pallas_call(
        _xpose_body,
        grid=(B, NUM_HEADS, nt),
        in_specs=[
            pl.BlockSpec((1, 1, TBLK, NBS, NKK), lambda b, h, t: (h, b, t, 0, 0)),
        ],
        out_specs=pl.BlockSpec(
            (1, 1, NBS, NKK, TBLK), lambda b, h, t: (h, b, 0, 0, t)
        ),
        out_shape=jax.ShapeDtypeStruct((NUM_HEADS, B, NBS, NKK, nbt), jnp.float32),
    )(a)


def kernel(x, indexes, weights, W_in, b_in):
    B, H, W, _ = x.shape
    nbh, nbw = H // BS, W // BS
    nbt = nbh * nbw
    num_tokens = H * W

    wq = W_in[:QDIM].T
    wk = jnp.pad(W_in[QDIM:].T, ((0, 0), (0, KPAD - QDIM)))
    bq = b_in[:QDIM].reshape(1, QDIM)
    bk = jnp.pad(b_in[QDIM:], (0, KPAD - QDIM)).reshape(1, KPAD)

    x2d = x.reshape(B * num_tokens, EMBED_DIM)
    q, k = _project(x2d, wq, wk, bq, bk)

    idx = (indexes + (jnp.arange(B, dtype=jnp.int32) * num_tokens)[:, None, None])
    idx = idx.reshape(-1)
    sel = _make_sc_gather(idx.shape[0], chunk=224)(k, idx)

    q5 = q.reshape(B, nbh, BS, W, QDIM)
    k5 = k.reshape(B, nbh, BS, W, KPAD)
    sel5 = sel.reshape(B, nbh, nbw, TOPK, KPAD)
    w5 = weights.reshape(B, nbh, nbw, TOPK, 1)

    att = _attention(q5, k5, sel5, w5, B, nbh, nbw)
    return jnp.transpose(_to_blk_minor(att, B, nbt), (0, 1, 4, 2, 3))
